# baseline (device time: 34281 ns/iter reference)
import functools

import jax
import jax.numpy as jnp
from jax import lax
from jax.experimental import pallas as pl
from jax.experimental.pallas import tpu as pltpu

N_DEV = 8
E_TOTAL = 16
E_LOC = 2
HOPS = 4


def kernel(x, router_W, route_idx, expert_W):
    n_tok, d = x.shape
    e_loc, _, h = expert_W.shape
    assert e_loc == E_LOC

    cw_js = [(0, 1), (0, 1), (0, 1), (0,)]
    ccw_js = [(0, 1), (0, 1), (0, 1), (1,)]

    def body(x_ref, rw_ref, idx_ref, ew_ref, out_ref,
             cw_ref, ccw_ref, cw_send, cw_recv, ccw_send, ccw_recv):
        my = lax.axis_index("i")
        left = lax.rem(my + N_DEV - 1, N_DEV)
        right = lax.rem(my + 1, N_DEV)

        barrier_sem = pltpu.get_barrier_semaphore()
        for nbr in (left, right):
            pl.semaphore_signal(
                barrier_sem, inc=1,
                device_id=(nbr,), device_id_type=pl.DeviceIdType.MESH,
            )
        pl.semaphore_wait(barrier_sem, 2)

        def cw_rdma(hop, j):
            return pltpu.make_async_remote_copy(
                src_ref=cw_ref.at[hop, j],
                dst_ref=cw_ref.at[hop + 1, j],
                send_sem=cw_send.at[hop, j],
                recv_sem=cw_recv.at[hop, j],
                device_id=(right,),
                device_id_type=pl.DeviceIdType.MESH,
            )

        def ccw_rdma(hop, j):
            return pltpu.make_async_remote_copy(
                src_ref=cw_ref.at[0, j] if hop == 0 else ccw_ref.at[hop - 1, j],
                dst_ref=ccw_ref.at[hop, j],
                send_sem=ccw_send.at[hop, j],
                recv_sem=ccw_recv.at[hop, j],
                device_id=(left,),
                device_id_type=pl.DeviceIdType.MESH,
            )

        for j in range(E_LOC):
            cw_ref[0, j] = ew_ref[j].astype(jnp.bfloat16)
            cw_rdma(0, j).start()
            ccw_rdma(0, j).start()

        xf = x_ref[:, :]
        scores = jnp.dot(xf, rw_ref[:, :], preferred_element_type=jnp.float32)
        s_max = jnp.max(scores, axis=-1, keepdims=True)
        probs = jnp.exp(scores - s_max)
        probs = probs / jnp.sum(probs, axis=-1, keepdims=True)

        col_ids = lax.broadcasted_iota(jnp.int32, (n_tok, E_TOTAL), 1)
        top2 = (idx_ref[:, 0:1] == col_ids) | (idx_ref[:, 1:2] == col_ids)
        wfull = jnp.where(top2, probs, 0.0)
        w = wfull / jnp.sum(wfull, axis=-1, keepdims=True)

        xb = xf.astype(jnp.bfloat16)

        def one_expert(wref, slot, j, src):
            e = src * E_LOC + j
            y = jnp.dot(xb, wref[slot, j], preferred_element_type=jnp.float32)
            wtok = jnp.sum(jnp.where(col_ids == e, w, 0.0),
                           axis=1, keepdims=True)
            return wtok * y

        out_ref[:, :] = one_expert(cw_ref, 0, 0, my) + one_expert(cw_ref, 0, 1, my)

        for hop in range(HOPS):
            for j in range(E_LOC):
                if j in cw_js[hop]:
                    cw_rdma(hop, j).wait_recv()
                    if hop + 1 < HOPS and j in cw_js[hop + 1]:
                        cw_rdma(hop + 1, j).start()
                if j in ccw_js[hop]:
                    ccw_rdma(hop, j).wait_recv()
                    if hop + 1 < HOPS and j in ccw_js[hop + 1]:
                        ccw_rdma(hop + 1, j).start()
            cw_src = lax.rem(my + N_DEV - 1 - hop, N_DEV)
            ccw_src = lax.rem(my + 1 + hop, N_DEV)
            acc = jnp.zeros((n_tok, h), jnp.float32)
            for j in cw_js[hop]:
                acc = acc + one_expert(cw_ref, hop + 1, j, cw_src)
            for j in ccw_js[hop]:
                acc = acc + one_expert(ccw_ref, hop, j, ccw_src)
            out_ref[:, :] = out_ref[:, :] + acc

        for hop in range(HOPS):
            for j in cw_js[hop]:
                cw_rdma(hop, j).wait_send()
            for j in ccw_js[hop]:
                ccw_rdma(hop, j).wait_send()

        @functools.partial(
            pl.run_scoped, second_barrier=pltpu.SemaphoreType.REGULAR
        )
        def _(second_barrier):
            for nbr in (left, right):
                pl.semaphore_signal(
                    second_barrier, inc=1,
                    device_id=(nbr,), device_id_type=pl.DeviceIdType.MESH,
                )
            pl.semaphore_wait(second_barrier, 2)

    return pl.pallas_call(
        body,
        out_shape=jax.ShapeDtypeStruct((n_tok, h), jnp.float32),
        in_specs=[
            pl.BlockSpec(memory_space=pltpu.VMEM),
            pl.BlockSpec(memory_space=pltpu.VMEM),
            pl.BlockSpec(memory_space=pltpu.VMEM),
            pl.BlockSpec(memory_space=pltpu.VMEM),
        ],
        out_specs=pl.BlockSpec(memory_space=pltpu.VMEM),
        scratch_shapes=[
            pltpu.VMEM((HOPS + 1, E_LOC, d, h), jnp.bfloat16),
            pltpu.VMEM((HOPS, E_LOC, d, h), jnp.bfloat16),
            pltpu.SemaphoreType.DMA((HOPS, E_LOC)),
            pltpu.SemaphoreType.DMA((HOPS, E_LOC)),
            pltpu.SemaphoreType.DMA((HOPS, E_LOC)),
            pltpu.SemaphoreType.DMA((HOPS, E_LOC)),
        ],
        compiler_params=pltpu.CompilerParams(collective_id=0),
    )(x, router_W, route_idx, expert_W)
